# baseline (speedup 1.0000x reference)
import functools

import jax
import jax.numpy as jnp
from jax import lax
from jax.experimental import pallas as pl
from jax.experimental.pallas import tpu as pltpu
from jax.experimental.pallas import tpu_sc as plsc

N_USERS = 1_000_000
N_ITEMS = 100_000
DIM = 32

_NC = 2
_NS = 16
_NW = _NC * _NS

_CHUNK = 480
_NB = 2

_U_SLOTS = -(-(-(-N_USERS // _CHUNK)) // _NW)

_I_BLOCK = 4000

_mesh = plsc.VectorSubcoreMesh(core_axis_name="c", subcore_axis_name="s")


@functools.partial(
    pl.kernel,
    out_type=jax.ShapeDtypeStruct((N_USERS, DIM), jnp.float32),
    mesh=_mesh,
    scratch_types=(
        [pltpu.VMEM((_CHUNK, DIM), jnp.float32) for _ in range(_NB)]
        + [pltpu.SemaphoreType.DMA for _ in range(2 * _NB)]
    ),
)
def _copy_user(u_in, u_out, *scratch):
    bufs = scratch[:_NB]
    rsems = scratch[_NB:2 * _NB]
    wsems = scratch[2 * _NB:]
    wid = lax.axis_index("s") * _NC + lax.axis_index("c")

    bases = []
    for t in range(_U_SLOTS):
        base = jnp.minimum((wid + t * _NW) * _CHUNK, N_USERS - _CHUNK)
        bases.append(pl.multiple_of(base, 8))
    n = len(bases)

    def read(j, b):
        return pltpu.async_copy(
            u_in.at[pl.ds(bases[j], _CHUNK)], bufs[b], rsems[b])

    def write(j, b):
        return pltpu.async_copy(
            bufs[b], u_out.at[pl.ds(bases[j], _CHUNK)], wsems[b])

    reads = [None] * n
    writes = [None] * n
    for b in range(min(_NB, n)):
        reads[b] = read(b, b)
    for j in range(n):
        b = j % _NB
        reads[j].wait()
        writes[j] = write(j, b)
        if j + _NB < n:
            writes[j].wait()
            reads[j + _NB] = read(j + _NB, b)
    for j in range(max(0, n - _NB), n):
        writes[j].wait()


def _tc_copy_body(src_ref, dst_ref):
    dst_ref[...] = src_ref[...]


def kernel(user_table, item_table):
    item_out = pl.pallas_call(
        _tc_copy_body,
        grid=(N_ITEMS // _I_BLOCK,),
        in_specs=[pl.BlockSpec((_I_BLOCK, DIM), lambda i: (i, 0))],
        out_specs=pl.BlockSpec((_I_BLOCK, DIM), lambda i: (i, 0)),
        out_shape=jax.ShapeDtypeStruct((N_ITEMS, DIM), jnp.float32),
    )(item_table)
    user_out = _copy_user(user_table)
    return user_out, item_out
